# baseline (speedup 1.0000x reference)
import functools

import jax
import jax.numpy as jnp
import numpy as np
from jax import lax
from jax.experimental import pallas as pl
from jax.experimental.pallas import tpu as pltpu
from jax.experimental.pallas import tpu_sc as plsc

V_PAD = 1024
_V_REAL = 1000
N_X = 16384
N_Y = 1024
N_YW = N_Y // 2

_NC, _NS = 2, 16
_NW = _NC * _NS
_CHUNK = 32


_COLA = np.array([32 * (w // 16) + (w % 16) for w in range(N_YW)], np.int32)
_COLB = _COLA + 16


def _round_bf16_bits(x):
    u = lax.bitcast_convert_type(x, jnp.int32)
    rnd = u + jnp.int32(0x7FFF) + (lax.shift_right_logical(u, 16) & jnp.int32(1))
    return lax.shift_right_logical(rnd, 16)


def _tc_table_body(gp_ref, iya_ref, iyb_ref, packed_ref):
    gp = gp_ref[...]
    rows = lax.broadcasted_iota(jnp.int32, (V_PAD, V_PAD), 0)
    cols = lax.broadcasted_iota(jnp.int32, (V_PAD, V_PAD), 1)
    sp = jnp.maximum(gp, 0.0) + jnp.log(1.0 + jnp.exp(-jnp.abs(gp)))
    valid = (rows < _V_REAL) & (cols < _V_REAL)
    L = jnp.where(valid & (cols < rows), gp,
                  jnp.where(valid & (cols == rows), sp, 0.0))
    gram = lax.dot_general(L, L, (((1,), (1,)), ((), ())),
                           preferred_element_type=jnp.float32)
    rows_h = lax.broadcasted_iota(jnp.int32, (V_PAD, N_YW), 0)
    oha = (rows_h == iya_ref[0, :][None, :]).astype(jnp.float32)
    ohb = (rows_h == iyb_ref[0, :][None, :]).astype(jnp.float32)
    sa = jnp.dot(gram, oha, preferred_element_type=jnp.float32)
    sb = jnp.dot(gram, ohb, preferred_element_type=jnp.float32)
    packed_ref[...] = _round_bf16_bits(sa) | lax.shift_left(
        _round_bf16_bits(sb), jnp.int32(16)
    )


_tc_table = pl.pallas_call(
    _tc_table_body,
    grid=(1,),
    in_specs=[
        pl.BlockSpec((V_PAD, V_PAD), lambda i: (0, 0)),
        pl.BlockSpec((1, N_YW), lambda i: (0, 0)),
        pl.BlockSpec((1, N_YW), lambda i: (0, 0)),
    ],
    out_specs=pl.BlockSpec((V_PAD, N_YW), lambda i: (0, 0)),
    out_shape=jax.ShapeDtypeStruct((V_PAD, N_YW), jnp.int32),
)


@functools.cache
def _make_sc_gather(n_rows):
    _BPW = n_rows // _NW
    _NCHUNK = _BPW // _CHUNK
    mesh = plsc.VectorSubcoreMesh(core_axis_name="c", subcore_axis_name="s")

    @functools.partial(
        pl.kernel,
        mesh=mesh,
        out_type=jax.ShapeDtypeStruct((n_rows, N_Y), jnp.int32),
        scratch_types=[
            pltpu.VMEM((_BPW,), jnp.int32),
            pltpu.VMEM((_CHUNK, N_YW), jnp.int32),
            pltpu.VMEM((_CHUNK, N_YW), jnp.int32),
            pltpu.VMEM((_CHUNK, N_Y), jnp.int32),
            pltpu.VMEM((_CHUNK, N_Y), jnp.int32),
            pltpu.SemaphoreType.DMA,
            pltpu.SemaphoreType.DMA,
            pltpu.SemaphoreType.DMA,
            pltpu.SemaphoreType.DMA,
        ],
    )
    def _sc_gather(
        table_hbm, idx_hbm, out_hbm, idx_v, p0, p1, f0, f1, g0, g1, s0, s1
    ):
        wid = lax.axis_index("s") * _NC + lax.axis_index("c")
        base = wid * _BPW
        pltpu.sync_copy(idx_hbm.at[pl.ds(base, _BPW)], idx_v)
        pbufs, fbufs, gsem, ssem = (p0, p1), (f0, f1), (g0, g1), (s0, s1)

        def start_gather(c):
            b = c & 1
            return pltpu.async_copy(
                table_hbm.at[idx_v.at[pl.ds(c * _CHUNK, _CHUNK)]], pbufs[b], gsem[b]
            )

        def upconvert(pref, fref):
            def row(r, carry):
                for c in range(N_YW // 16):
                    x = pref[r, pl.ds(c * 16, 16)]
                    fref[r, pl.ds(c * 32, 16)] = x << 16
                    fref[r, pl.ds(c * 32 + 16, 16)] = x & jnp.int32(-65536)
                return carry

            lax.fori_loop(0, _CHUNK, row, 0, unroll=False)

        gathers = {0: start_gather(0), 1: start_gather(1)}
        scatters = {}
        for c in range(_NCHUNK):
            b = c & 1
            gathers[c].wait()
            if c >= 2:
                scatters[c - 2].wait()
            upconvert(pbufs[b], fbufs[b])
            scatters[c] = pltpu.async_copy(
                fbufs[b], out_hbm.at[pl.ds(base + c * _CHUNK, _CHUNK)], ssem[b]
            )
            if c + 2 < _NCHUNK:
                gathers[c + 2] = start_gather(c + 2)
        scatters[_NCHUNK - 2].wait()
        scatters[_NCHUNK - 1].wait()

    return _sc_gather


def kernel(gram_param, idx_X, idx_Y):
    iyf = idx_Y.reshape(-1).astype(jnp.int32)
    iya = iyf[_COLA].reshape(1, -1)
    iyb = iyf[_COLB].reshape(1, -1)
    packed = _tc_table(gram_param, iya, iyb)
    ix = idx_X.reshape(-1).astype(jnp.int32)
    raw = _make_sc_gather(ix.shape[0])(packed, ix)
    return lax.bitcast_convert_type(raw, jnp.float32)


# speedup vs baseline: 2.1162x; 2.1162x over previous
import functools

import jax
import jax.numpy as jnp
from jax import lax
from jax.experimental import pallas as pl
from jax.experimental.pallas import tpu as pltpu
from jax.experimental.pallas import tpu_sc as plsc

V_PAD = 1024
_V_REAL = 1000
N_X = 16384
N_Y = 1024

_NC, _NS = 2, 16
_NW = _NC * _NS
_CHUNK = 32


def _tc_table_body(gp_ref, iy_ref, small_ref):
    gp = gp_ref[...]
    rows = lax.broadcasted_iota(jnp.int32, (V_PAD, V_PAD), 0)
    cols = lax.broadcasted_iota(jnp.int32, (V_PAD, V_PAD), 1)
    sp = jnp.maximum(gp, 0.0) + jnp.log(1.0 + jnp.exp(-jnp.abs(gp)))
    valid = (rows < _V_REAL) & (cols < _V_REAL)
    L = jnp.where(valid & (cols < rows), gp,
                  jnp.where(valid & (cols == rows), sp, 0.0))
    gram = lax.dot_general(L, L, (((1,), (1,)), ((), ())),
                           preferred_element_type=jnp.float32)
    iy = iy_ref[0, :]
    onehot = (rows == iy[None, :]).astype(jnp.float32)
    small_ref[...] = jnp.dot(gram, onehot, preferred_element_type=jnp.float32)


_tc_table = pl.pallas_call(
    _tc_table_body,
    grid=(1,),
    in_specs=[
        pl.BlockSpec((V_PAD, V_PAD), lambda i: (0, 0)),
        pl.BlockSpec((1, N_Y), lambda i: (0, 0)),
    ],
    out_specs=pl.BlockSpec((V_PAD, N_Y), lambda i: (0, 0)),
    out_shape=jax.ShapeDtypeStruct((V_PAD, N_Y), jnp.float32),
)


@functools.cache
def _make_sc_gather(n_rows):
    _BPW = n_rows // _NW
    _NCHUNK = _BPW // _CHUNK
    mesh = plsc.VectorSubcoreMesh(core_axis_name="c", subcore_axis_name="s")

    @functools.partial(
        pl.kernel,
        mesh=mesh,
        out_type=jax.ShapeDtypeStruct((n_rows, N_Y), jnp.float32),
        scratch_types=[
            pltpu.VMEM((_BPW,), jnp.int32),
            pltpu.VMEM((_CHUNK, N_Y), jnp.float32),
            pltpu.VMEM((_CHUNK, N_Y), jnp.float32),
            pltpu.SemaphoreType.DMA,
            pltpu.SemaphoreType.DMA,
            pltpu.SemaphoreType.DMA,
            pltpu.SemaphoreType.DMA,
        ],
    )
    def _sc_gather(table_hbm, idx_hbm, out_hbm, idx_v, b0, b1, g0, g1, s0, s1):
        sid = lax.axis_index("s")
        wid = sid * _NC + lax.axis_index("c")
        base = wid * _BPW
        pltpu.sync_copy(idx_hbm.at[pl.ds(base, _BPW)], idx_v)
        bufs, gsem, ssem = (b0, b1), (g0, g1), (s0, s1)

        def start_gather(c):
            b = c & 1
            return pltpu.async_copy(
                table_hbm.at[idx_v.at[pl.ds(c * _CHUNK, _CHUNK)]], bufs[b], gsem[b]
            )

        gathers = {0: start_gather(0), 1: start_gather(1)}
        scatters = {}
        for c in range(_NCHUNK):
            b = c & 1
            gathers[c].wait()
            scatters[c] = pltpu.async_copy(
                bufs[b], out_hbm.at[pl.ds(base + c * _CHUNK, _CHUNK)], ssem[b]
            )
            if c + 2 < _NCHUNK:
                scatters[c].wait()
                gathers[c + 2] = start_gather(c + 2)
        scatters[_NCHUNK - 2].wait()
        scatters[_NCHUNK - 1].wait()

    return _sc_gather


def _one_device(gram_param, idx_X, idx_Y):
    iy = idx_Y.reshape(1, -1).astype(jnp.int32)
    small = _tc_table(gram_param, iy)
    ix = idx_X.reshape(-1).astype(jnp.int32)
    return _make_sc_gather(ix.shape[0])(small, ix)


def kernel(gram_param, idx_X, idx_Y):
    return _one_device(gram_param, idx_X, idx_Y)
